# Initial kernel scaffold; baseline (speedup 1.0000x reference)
import jax
import jax.numpy as jnp
from jax.experimental import pallas as pl


def kernel(x, separated_src_idx, unique):
    raise NotImplementedError("write your pallas kernel here")


# speedup vs baseline: 3.1395x; 3.1395x over previous
import functools

import jax
import jax.numpy as jnp
from jax import lax
from jax.experimental import pallas as pl
from jax.experimental.pallas import tpu as pltpu
from jax.experimental.pallas import tpu_sc as plsc

V = 40962
VP = 40968
E = 40960
U = 3 * E
NCH = 32
ROWS = 4 * NCH
OUT_W = V + U

NC = 2
NS = 16
NW = NC * NS
RPW = ROWS // NW

CH = 2048
IDXB = CH + 8


def _sc_shuffle(xf, l0p, l1p):
    mesh = plsc.VectorSubcoreMesh(core_axis_name="c", subcore_axis_name="s")

    @functools.partial(
        pl.kernel,
        mesh=mesh,
        out_type=jax.ShapeDtypeStruct((ROWS * OUT_W,), jnp.float32),
        compiler_params=pltpu.CompilerParams(needs_layout_passes=False),
        scratch_types=[
            pltpu.VMEM((3 * VP,), jnp.float32),
            pltpu.VMEM((IDXB,), jnp.int32),
            pltpu.VMEM((IDXB,), jnp.int32),
            pltpu.VMEM((CH + 16,), jnp.float32),
            pltpu.VMEM((24,), jnp.float32),
        ],
    )
    def k(xf_hbm, l0_hbm, l1_hbm, out_hbm, srcbuf, i0buf, i1buf, obuf, nbuf):
        wid = lax.axis_index("s") * NC + lax.axis_index("c")
        r0 = wid * RPW

        def gather16(joff, a):
            ia = i0buf[pl.ds(joff, 16)] + a
            ib = i1buf[pl.ds(joff, 16)] + a
            va = plsc.load_gather(srcbuf, [ia])
            vb = plsc.load_gather(srcbuf, [ib])
            return (va + vb) * 0.5

        for rr in range(RPW):
            r = r0 + rr
            b = r // NCH
            c = r % NCH
            a = 2 * rr
            delta = (8 - 2 * rr) % 8
            phi = (6 - 2 * rr) % 8
            B = r * OUT_W
            C = (b * 128 + c) * V

            for g in range(3):
                S = (b * 128 + (g + 1) * NCH + c) * V
                pltpu.sync_copy(
                    xf_hbm.at[pl.ds(pl.multiple_of(S - a, 8), V + a)],
                    srcbuf.at[pl.ds(g * VP, V + a)],
                )

            nfull, rem = divmod(V - delta, CH)

            def cbody(m, _):
                off = pl.multiple_of(C + delta + CH * m, 8)
                oo = pl.multiple_of(B + delta + CH * m, 8)
                pltpu.sync_copy(xf_hbm.at[pl.ds(off, CH)], obuf.at[pl.ds(0, CH)])
                pltpu.sync_copy(obuf.at[pl.ds(0, CH)], out_hbm.at[pl.ds(oo, CH)])
                return 0

            lax.fori_loop(0, nfull, cbody, 0)

            if rem + phi > 0:
                start = delta + nfull * CH
                if rem > 0:
                    pltpu.sync_copy(
                        xf_hbm.at[pl.ds(pl.multiple_of(C + start, 8), rem)],
                        obuf.at[pl.ds(0, rem)],
                    )
                if phi > 0:
                    pltpu.sync_copy(l0_hbm.at[pl.ds(0, 16)], i0buf.at[pl.ds(0, 16)])
                    pltpu.sync_copy(l1_hbm.at[pl.ds(0, 16)], i1buf.at[pl.ds(0, 16)])
                    obuf[pl.ds(rem, 16)] = gather16(0, a)
                pltpu.sync_copy(
                    obuf.at[pl.ds(0, rem + phi)],
                    out_hbm.at[pl.ds(pl.multiple_of(B + start, 8), rem + phi)],
                )

            mf = (U - phi) // CH

            def fbody(m, _):
                joff = pl.multiple_of(CH * m, 8)
                pltpu.sync_copy(l0_hbm.at[pl.ds(joff, IDXB)], i0buf.at[pl.ds(0, IDXB)])
                pltpu.sync_copy(l1_hbm.at[pl.ds(joff, IDXB)], i1buf.at[pl.ds(0, IDXB)])

                def gbody(kk, _):
                    base = kk * 128
                    for uu in range(8):
                        s = base + 16 * uu
                        obuf[pl.ds(s, 16)] = gather16(phi + s, a)
                    return 0

                lax.fori_loop(0, CH // 128, gbody, 0)
                oo = pl.multiple_of(B + V + phi + CH * m, 8)
                pltpu.sync_copy(obuf.at[pl.ds(0, CH)], out_hbm.at[pl.ds(oo, CH)])
                return 0

            lax.fori_loop(0, mf, fbody, 0)

            if phi > 0:
                jal = mf * CH
                t2 = CH - phi
                pltpu.sync_copy(l0_hbm.at[pl.ds(jal, IDXB)], i0buf.at[pl.ds(0, IDXB)])
                pltpu.sync_copy(l1_hbm.at[pl.ds(jal, IDXB)], i1buf.at[pl.ds(0, IDXB)])
                for gg in range(CH // 16):
                    s = 16 * gg
                    obuf[pl.ds(s, 16)] = gather16(phi + s, a)
                rn = r + 1
                bn = rn // NCH
                cn = rn % NCH
                Cn = (bn * 128 + cn) * V
                chi = 2 * (rr + 1)
                pltpu.sync_copy(
                    xf_hbm.at[pl.ds(pl.multiple_of(Cn - chi, 8), 16)],
                    nbuf.at[pl.ds(0, 16)],
                )
                obuf[pl.ds(t2, 16)] = nbuf[pl.ds(chi, 16)]
                oo = pl.multiple_of(B + V + jal + phi, 8)
                pltpu.sync_copy(obuf.at[pl.ds(0, CH)], out_hbm.at[pl.ds(oo, CH)])

    return k(xf, l0p, l1p)


def kernel(x, separated_src_idx, unique):
    u = unique
    grp = (u >= E).astype(jnp.int32) + (u >= 2 * E).astype(jnp.int32)
    ssi_flat = separated_src_idx.reshape(-1)
    v0 = jnp.take(ssi_flat, 2 * u)
    v1 = jnp.take(ssi_flat, 2 * u + 1)
    pad = jnp.zeros((8,), jnp.int32)
    l0p = jnp.concatenate([grp * VP + v0, pad])
    l1p = jnp.concatenate([grp * VP + v1, pad])

    xf = x.reshape(-1)
    out = _sc_shuffle(xf, l0p, l1p)
    return out.reshape(4, NCH, OUT_W)


# speedup vs baseline: 3.4724x; 1.1060x over previous
import functools

import jax
import jax.numpy as jnp
from jax import lax
from jax.experimental import pallas as pl
from jax.experimental.pallas import tpu as pltpu
from jax.experimental.pallas import tpu_sc as plsc

V = 40962
VP = 40968
E = 40960
U = 3 * E
NCH = 32
ROWS = 4 * NCH
OUT_W = V + U

NC = 2
NS = 16
NW = NC * NS
RPW = ROWS // NW

CH = 1024
NFC = U // CH
IDXB = CH + 8
OB = CH + 16


def _sc_shuffle(xf, l0p, l1p):
    mesh = plsc.VectorSubcoreMesh(core_axis_name="c", subcore_axis_name="s")

    @functools.partial(
        pl.kernel,
        mesh=mesh,
        out_type=jax.ShapeDtypeStruct((ROWS * OUT_W,), jnp.float32),
        compiler_params=pltpu.CompilerParams(needs_layout_passes=False),
        scratch_types=[
            pltpu.VMEM((3 * VP,), jnp.float32),
            pltpu.VMEM((2 * IDXB,), jnp.int32),
            pltpu.VMEM((2 * IDXB,), jnp.int32),
            pltpu.VMEM((2 * OB,), jnp.float32),
            pltpu.VMEM((24,), jnp.float32),
            pltpu.SemaphoreType.DMA((2,)),
            pltpu.SemaphoreType.DMA((2,)),
            pltpu.SemaphoreType.DMA,
            pltpu.SemaphoreType.DMA((2,)),
        ],
    )
    def k(xf_hbm, l0_hbm, l1_hbm, out_hbm, srcbuf, i0r, i1r, orr, nbuf,
          semI, semO, semS, semC):
        wid = lax.axis_index("s") * NC + lax.axis_index("c")
        r0 = wid * RPW

        def issue_idx(t, p):
            po = pl.multiple_of(p * IDXB, 8)
            off = pl.multiple_of(t * CH, 8)
            pltpu.async_copy(l0_hbm.at[pl.ds(off, IDXB)],
                             i0r.at[pl.ds(po, IDXB)], semI.at[p])
            pltpu.async_copy(l1_hbm.at[pl.ds(off, IDXB)],
                             i1r.at[pl.ds(po, IDXB)], semI.at[p])

        def wait_idx(t, p):
            po = pl.multiple_of(p * IDXB, 8)
            off = pl.multiple_of(t * CH, 8)
            pltpu.make_async_copy(l0_hbm.at[pl.ds(off, IDXB)],
                                  i0r.at[pl.ds(po, IDXB)], semI.at[p]).wait()
            pltpu.make_async_copy(l1_hbm.at[pl.ds(off, IDXB)],
                                  i1r.at[pl.ds(po, IDXB)], semI.at[p]).wait()

        def gather16(ref0off, a, p):
            po = p * IDXB
            ia = i0r[pl.ds(po + ref0off, 16)] + a
            ib = i1r[pl.ds(po + ref0off, 16)] + a
            va = plsc.load_gather(srcbuf, [ia])
            vb = plsc.load_gather(srcbuf, [ib])
            return (va + vb) * 0.5

        for rr in range(RPW):
            r = r0 + rr
            b = r // NCH
            c = r % NCH
            a = 2 * rr
            delta = (8 - 2 * rr) % 8
            phi = (6 - 2 * rr) % 8
            rem8 = (2 * rr + 2) % 8
            B = r * OUT_W
            C = (b * 128 + c) * V

            def stage(g):
                S = (b * 128 + (g + 1) * NCH + c) * V
                return pltpu.make_async_copy(
                    xf_hbm.at[pl.ds(pl.multiple_of(S - a, 8), V + a)],
                    srcbuf.at[pl.ds(g * VP, V + a)], semS)

            stage(1).start()
            stage(2).start()

            cs = V - rem8 - delta
            HB = 20480
            sizes = (HB, cs - HB)

            def cin(h):
                off = pl.multiple_of(C + delta + HB * h, 8)
                return pltpu.make_async_copy(
                    xf_hbm.at[pl.ds(off, sizes[h])],
                    srcbuf.at[pl.ds(HB * h, sizes[h])], semC.at[h])

            def cout(h):
                oo = pl.multiple_of(B + delta + HB * h, 8)
                return pltpu.make_async_copy(
                    srcbuf.at[pl.ds(HB * h, sizes[h])],
                    out_hbm.at[pl.ds(oo, sizes[h])], semO.at[h])

            cin(0).start()
            cin(1).start()
            cin(0).wait()
            cout(0).start()
            cin(1).wait()
            cout(1).start()
            cout(0).wait()
            cout(1).wait()

            stage(0).start()
            issue_idx(0, 0)
            issue_idx(1, 1)
            for g in range(3):
                stage(g).wait()

            if rem8 + phi > 0:
                pltpu.sync_copy(
                    xf_hbm.at[pl.ds(pl.multiple_of(C + V - rem8, 8), rem8)],
                    nbuf.at[pl.ds(0, rem8)])
                wait_idx(0, 0)
                nbuf[pl.ds(rem8, 16)] = gather16(0, a, 0)
                pltpu.sync_copy(
                    nbuf.at[pl.ds(0, rem8 + phi)],
                    out_hbm.at[pl.ds(pl.multiple_of(B + V - rem8, 8), rem8 + phi)])
                seam_waited = True
            else:
                seam_waited = False

            def _when(cond, fn):
                if isinstance(cond, bool):
                    if cond:
                        fn()
                else:
                    pl.when(cond)(fn)

            def fchunk(t, p, first_wait, tail):
                if first_wait:
                    wait_idx(t, p)

                def _wait_prev_out():
                    po2 = pl.multiple_of(p * OB, 8)
                    oo2 = pl.multiple_of(B + V + phi + CH * (t - 2), 8)
                    pltpu.make_async_copy(orr.at[pl.ds(po2, CH)],
                                          out_hbm.at[pl.ds(oo2, CH)],
                                          semO.at[p]).wait()

                _when(t >= 2, _wait_prev_out)

                po = pl.multiple_of(p * OB, 8)
                for gg in range(CH // 16):
                    s = 16 * gg
                    orr[pl.ds(po + s, 16)] = gather16(phi + s, a, p)
                if tail and phi > 0:
                    rn = r + 1
                    Cn = ((rn // NCH) * 128 + rn % NCH) * V
                    chi = 2 * (rr + 1)
                    pltpu.sync_copy(
                        xf_hbm.at[pl.ds(pl.multiple_of(Cn - chi, 8), 16)],
                        nbuf.at[pl.ds(0, 16)])
                    orr[pl.ds(po + CH - phi, 16)] = nbuf[pl.ds(chi, 16)]
                oo = pl.multiple_of(B + V + phi + CH * t, 8)
                pltpu.async_copy(orr.at[pl.ds(po, CH)],
                                 out_hbm.at[pl.ds(oo, CH)], semO.at[p])
                if not tail:
                    _when(t <= NFC - 3, lambda: issue_idx(t + 2, p))

            def fbody(t, _):
                p = lax.rem(t, 2)
                fchunk(t, p, True, False)
                return 0

            if seam_waited:
                fchunk(0, 0, False, False)
                lax.fori_loop(1, NFC - 1, fbody, 0)
            else:
                lax.fori_loop(0, NFC - 1, fbody, 0)
            fchunk(NFC - 1, (NFC - 1) % 2, True, True)

            for t in (NFC - 2, NFC - 1):
                p = t % 2
                po = pl.multiple_of(p * OB, 8)
                oo = pl.multiple_of(B + V + phi + CH * t, 8)
                pltpu.make_async_copy(orr.at[pl.ds(po, CH)],
                                      out_hbm.at[pl.ds(oo, CH)],
                                      semO.at[p]).wait()

    return k(xf, l0p, l1p)


def kernel(x, separated_src_idx, unique):
    u = unique
    grp = (u >= E).astype(jnp.int32) + (u >= 2 * E).astype(jnp.int32)
    ssi_flat = separated_src_idx.reshape(-1)
    v0 = jnp.take(ssi_flat, 2 * u)
    v1 = jnp.take(ssi_flat, 2 * u + 1)
    pad = jnp.zeros((8,), jnp.int32)
    l0p = jnp.concatenate([grp * VP + v0, pad])
    l1p = jnp.concatenate([grp * VP + v1, pad])

    xf = x.reshape(-1)
    out = _sc_shuffle(xf, l0p, l1p)
    return out.reshape(4, NCH, OUT_W)


# speedup vs baseline: 9.0820x; 2.6155x over previous
import functools

import jax
import jax.numpy as jnp
from jax import lax
from jax.experimental import pallas as pl
from jax.experimental.pallas import tpu as pltpu
from jax.experimental.pallas import tpu_sc as plsc

V = 40962
E = 40960
U = 3 * E
OUT_W = V + U
QT = 1281
NCOPY = 320
NFACE = 960
SEAM = 320

NC = 2
NS = 16
NW = NC * NS
CPW = NCOPY // NW
FPW = NFACE // NW

LP_LEN = 2 + U + 126


def _sc_shuffle(T, l0p, l1p):
    mesh = plsc.VectorSubcoreMesh(core_axis_name="c", subcore_axis_name="s")

    @functools.partial(
        pl.kernel,
        mesh=mesh,
        out_type=jax.ShapeDtypeStruct((4 * 4 * QT * 8 * 128,), jnp.float32),
        compiler_params=pltpu.CompilerParams(needs_layout_passes=False),
        scratch_types=[
            pltpu.VMEM((128, 128), jnp.float32),
            pltpu.VMEM((128, 128), jnp.float32),
            pltpu.VMEM((128,), jnp.int32),
            pltpu.VMEM((128,), jnp.int32),
            pltpu.VMEM((16384,), jnp.float32),
            pltpu.VMEM((16384,), jnp.float32),
            pltpu.SemaphoreType.DMA,
            pltpu.SemaphoreType.DMA((2,)),
        ],
    )
    def k(t_hbm, l0_hbm, l1_hbm, out_hbm, A, B, i0v, i1v, oA, oB, semG, semO):
        wid = lax.axis_index("s") * NC + lax.axis_index("c")
        iota = lax.iota(jnp.int32, 16)
        dvec = []
        svec = []
        for phi in range(16):
            d = jnp.bitwise_and(iota + phi, 15)
            dvec.append(d)
            svec.append(iota + (d >> 3) * 1024 + jnp.bitwise_and(d, 7) * 128)

        def transpose_col(obuf, pair, seam):
            def _body(fi, _):
                b = (fi >> 4) & 3
                c0i = (fi >> 3) & 1
                kb = fi & 7
                k0 = kb * 16
                rowv = k0 + iota
                bc0 = b * 32 + c0i * 16
                sbase = (b * 4 + c0i * 2) * 1024 + k0
                for phi in range(16):
                    colv = bc0 + dvec[phi]
                    va = plsc.load_gather(A, [rowv, colv])
                    if pair:
                        vb = plsc.load_gather(B, [rowv, colv])
                        val = (va + vb) * 0.5
                    else:
                        val = va
                    if seam:
                        plsc.store_scatter(obuf, [sbase + svec[phi]], val,
                                           mask=rowv >= 2)
                    else:
                        plsc.store_scatter(obuf, [sbase + svec[phi]], val)
                return 0

            lax.fori_loop(0, 64, _body, 0)

        def emit_tiles(obuf, qt, p):
            for t16 in range(16):
                pos = pl.multiple_of((t16 * QT + qt) * 1024, 8)
                pltpu.async_copy(obuf.at[pl.ds(t16 * 1024, 1024)],
                                 out_hbm.at[pl.ds(pos, 1024)], semO.at[p])

        def drain_tiles(obuf, qt, p):
            for t16 in range(16):
                pos = pl.multiple_of((t16 * QT + qt) * 1024, 8)
                pltpu.make_async_copy(obuf.at[pl.ds(t16 * 1024, 1024)],
                                      out_hbm.at[pl.ds(pos, 1024)],
                                      semO.at[p]).wait()

        def gather_rows(qt):
            w0 = pl.multiple_of(qt * 128 - 40960, 8)
            pltpu.sync_copy(l0_hbm.at[pl.ds(w0, 128)], i0v)
            pltpu.sync_copy(l1_hbm.at[pl.ds(w0, 128)], i1v)
            pltpu.async_copy(t_hbm.at[i0v], A, semG).wait()
            pltpu.async_copy(t_hbm.at[i1v], B, semG).wait()

        def copy_col(qt, obuf, p, drain):
            r0 = pl.multiple_of(qt * 128, 8)
            pltpu.sync_copy(t_hbm.at[pl.ds(r0, 128), :], A)
            if drain:
                drain_tiles(obuf, qt, p)
            transpose_col(obuf, False, False)
            emit_tiles(obuf, qt, p)

        def face_col(qt, obuf, p, drain, seam=False):
            if seam:
                r0 = pl.multiple_of(qt * 128, 8)
                pltpu.sync_copy(t_hbm.at[pl.ds(r0, 128), :], A)
                if drain:
                    drain_tiles(obuf, qt, p)
                transpose_col(obuf, False, False)
                gather_rows(qt)
                transpose_col(obuf, True, True)
            else:
                gather_rows(qt)
                if drain:
                    drain_tiles(obuf, qt, p)
                transpose_col(obuf, True, False)
            emit_tiles(obuf, qt, p)

        cbase = wid * CPW
        copy_col(cbase, oA, 0, False)
        copy_col(cbase + 1, oB, 1, False)

        def cpair(t, _):
            q0 = cbase + 2 * t
            copy_col(q0, oA, 0, True)
            copy_col(q0 + 1, oB, 1, True)
            return 0

        lax.fori_loop(1, CPW // 2, cpair, 0)

        fbase = 321 + wid * FPW

        def fpair(t, _):
            q0 = fbase + 2 * t
            face_col(q0, oA, 0, True)
            face_col(q0 + 1, oB, 1, True)
            return 0

        lax.fori_loop(0, FPW // 2, fpair, 0)

        drain_tiles(oA, fbase + FPW - 2, 0)
        drain_tiles(oB, fbase + FPW - 1, 1)

        @pl.when(wid == 0)
        def _():
            face_col(SEAM, oA, 0, False, seam=True)
            drain_tiles(oA, SEAM, 0)

    return k(T, l0p, l1p)


def kernel(x, separated_src_idx, unique):
    u = unique
    grp1 = 1 + (u >= E).astype(jnp.int32) + (u >= 2 * E).astype(jnp.int32)
    ssi_flat = separated_src_idx.reshape(-1)
    v0 = jnp.take(ssi_flat, 2 * u)
    v1 = jnp.take(ssi_flat, 2 * u + 1)
    z2 = jnp.zeros((2,), jnp.int32)
    zt = jnp.zeros((126,), jnp.int32)
    l0p = jnp.concatenate([z2, grp1 * V + v0, zt])
    l1p = jnp.concatenate([z2, grp1 * V + v1, zt])

    T = x.reshape(4, 4, 32, V).transpose(1, 3, 0, 2).reshape(4 * V, 128)

    phy = _sc_shuffle(T, l0p, l1p)
    out = (
        phy.reshape(4, 4, QT, 8, 128)
        .transpose(0, 1, 3, 2, 4)
        .reshape(4, 32, QT * 128)[:, :, :OUT_W]
    )
    return out


# speedup vs baseline: 10.0507x; 1.1067x over previous
import functools

import jax
import jax.numpy as jnp
from jax import lax
from jax.experimental import pallas as pl
from jax.experimental.pallas import tpu as pltpu
from jax.experimental.pallas import tpu_sc as plsc

V = 40962
E = 40960
U = 3 * E
OUT_W = V + U
QT = 1281
NCOPY = 320
NFACE = 960
SEAM = 320

NC = 2
NS = 16
NW = NC * NS
CPW = NCOPY // NW
FPW = NFACE // NW

LP_LEN = 2 + U + 126


def _sc_shuffle(T, l0p, l1p):
    mesh = plsc.VectorSubcoreMesh(core_axis_name="c", subcore_axis_name="s")

    @functools.partial(
        pl.kernel,
        mesh=mesh,
        out_type=jax.ShapeDtypeStruct((4 * 4 * QT * 8 * 128,), jnp.float32),
        compiler_params=pltpu.CompilerParams(needs_layout_passes=False),
        scratch_types=[
            pltpu.VMEM((128, 128), jnp.float32),
            pltpu.VMEM((128, 128), jnp.float32),
            pltpu.VMEM((128, 128), jnp.float32),
            pltpu.VMEM((128, 128), jnp.float32),
            pltpu.VMEM((128,), jnp.int32),
            pltpu.VMEM((128,), jnp.int32),
            pltpu.VMEM((128,), jnp.int32),
            pltpu.VMEM((128,), jnp.int32),
            pltpu.VMEM((16384,), jnp.float32),
            pltpu.VMEM((16384,), jnp.float32),
            pltpu.SemaphoreType.DMA((2,)),
            pltpu.SemaphoreType.DMA((2,)),
        ],
    )
    def k(t_hbm, l0_hbm, l1_hbm, out_hbm, A0, B0, A1, B1, j00, j10, j01, j11,
          oA, oB, semG, semO):
        A, B, i0v, i1v = A0, B0, j00, j10
        wid = lax.axis_index("s") * NC + lax.axis_index("c")
        iota = lax.iota(jnp.int32, 16)
        dvec = []
        svec = []
        for phi in range(16):
            d = jnp.bitwise_and(iota + phi, 15)
            dvec.append(d)
            svec.append(iota + (d >> 3) * 1024 + jnp.bitwise_and(d, 7) * 128)

        def transpose_col(obuf, pair, seam, Ax, Bx):
            def _body(fi, _):
                b = (fi >> 4) & 3
                c0i = (fi >> 3) & 1
                kb = fi & 7
                k0 = kb * 16
                rowv = k0 + iota
                bc0 = b * 32 + c0i * 16
                sbase = (b * 4 + c0i * 2) * 1024 + k0
                for phi in range(16):
                    colv = bc0 + dvec[phi]
                    va = plsc.load_gather(Ax, [rowv, colv])
                    if pair:
                        vb = plsc.load_gather(Bx, [rowv, colv])
                        val = (va + vb) * 0.5
                    else:
                        val = va
                    if seam:
                        plsc.store_scatter(obuf, [sbase + svec[phi]], val,
                                           mask=rowv >= 2)
                    else:
                        plsc.store_scatter(obuf, [sbase + svec[phi]], val)
                return 0

            lax.fori_loop(0, 64, _body, 0)

        def emit_tiles(obuf, qt, p):
            for t16 in range(16):
                pos = pl.multiple_of((t16 * QT + qt) * 1024, 8)
                pltpu.async_copy(obuf.at[pl.ds(t16 * 1024, 1024)],
                                 out_hbm.at[pl.ds(pos, 1024)], semO.at[p])

        def drain_tiles(obuf, qt, p):
            for t16 in range(16):
                pos = pl.multiple_of((t16 * QT + qt) * 1024, 8)
                pltpu.make_async_copy(obuf.at[pl.ds(t16 * 1024, 1024)],
                                      out_hbm.at[pl.ds(pos, 1024)],
                                      semO.at[p]).wait()

        sets = ((A0, B0, j00, j10), (A1, B1, j01, j11))

        def issue_gather(qt, si):
            Ax, Bx, jx0, jx1 = sets[si]
            w0 = pl.multiple_of(qt * 128 - 40960, 8)
            pltpu.sync_copy(l0_hbm.at[pl.ds(w0, 128)], jx0)
            pltpu.sync_copy(l1_hbm.at[pl.ds(w0, 128)], jx1)
            pltpu.async_copy(t_hbm.at[jx0], Ax, semG.at[si])
            pltpu.async_copy(t_hbm.at[jx1], Bx, semG.at[si])

        def wait_gather(si):
            Ax, Bx, jx0, jx1 = sets[si]
            pltpu.make_async_copy(t_hbm.at[jx0], Ax, semG.at[si]).wait()
            pltpu.make_async_copy(t_hbm.at[jx1], Bx, semG.at[si]).wait()

        def copy_col(qt, obuf, p, drain):
            r0 = pl.multiple_of(qt * 128, 8)
            pltpu.sync_copy(t_hbm.at[pl.ds(r0, 128), :], A)
            if drain:
                drain_tiles(obuf, qt, p)
            transpose_col(obuf, False, False, A, B)
            emit_tiles(obuf, qt, p)

        cbase = wid * CPW
        copy_col(cbase, oA, 0, False)
        copy_col(cbase + 1, oB, 1, False)

        def cpair(t, _):
            q0 = cbase + 2 * t
            copy_col(q0, oA, 0, True)
            copy_col(q0 + 1, oB, 1, True)
            return 0

        lax.fori_loop(1, CPW // 2, cpair, 0)

        fbase = 321 + wid * FPW
        issue_gather(fbase, 0)

        def fpair(t, _):
            q0 = fbase + 2 * t
            issue_gather(q0 + 1, 1)
            wait_gather(0)
            drain_tiles(oA, q0, 0)
            transpose_col(oA, True, False, A0, B0)
            emit_tiles(oA, q0, 0)

            @pl.when(t <= FPW // 2 - 2)
            def _():
                issue_gather(q0 + 2, 0)

            wait_gather(1)
            drain_tiles(oB, q0 + 1, 1)
            transpose_col(oB, True, False, A1, B1)
            emit_tiles(oB, q0 + 1, 1)
            return 0

        lax.fori_loop(0, FPW // 2, fpair, 0)

        drain_tiles(oA, fbase + FPW - 2, 0)
        drain_tiles(oB, fbase + FPW - 1, 1)

        @pl.when(wid == 0)
        def _():
            r0 = pl.multiple_of(SEAM * 128, 8)
            pltpu.sync_copy(t_hbm.at[pl.ds(r0, 128), :], A)
            transpose_col(oA, False, False, A, B)
            issue_gather(SEAM, 0)
            wait_gather(0)
            transpose_col(oA, True, True, A, B)
            emit_tiles(oA, SEAM, 0)
            drain_tiles(oA, SEAM, 0)

    return k(T, l0p, l1p)


def kernel(x, separated_src_idx, unique):
    u = unique
    grp1 = 1 + (u >= E).astype(jnp.int32) + (u >= 2 * E).astype(jnp.int32)
    ssi_flat = separated_src_idx.reshape(-1)
    v0 = jnp.take(ssi_flat, 2 * u)
    v1 = jnp.take(ssi_flat, 2 * u + 1)
    z2 = jnp.zeros((2,), jnp.int32)
    zt = jnp.zeros((126,), jnp.int32)
    l0p = jnp.concatenate([z2, grp1 * V + v0, zt])
    l1p = jnp.concatenate([z2, grp1 * V + v1, zt])

    T = x.reshape(4, 4, 32, V).transpose(1, 3, 0, 2).reshape(4 * V, 128)

    phy = _sc_shuffle(T, l0p, l1p)
    out = (
        phy.reshape(4, 4, QT, 8, 128)
        .transpose(0, 1, 3, 2, 4)
        .reshape(4, 32, QT * 128)[:, :, :OUT_W]
    )
    return out
